# baseline (speedup 1.0000x reference)
import jax
import jax.numpy as jnp
from jax.experimental import pallas as pl
from jax.experimental.pallas import tpu as pltpu

B, N, F = 4, 2048, 64
NUM_HIDDEN = 5
CH = 512
NSLOT = 4
NCH = N // CH


def _soft(s, a):
    return jnp.where(s > a, s - a, jnp.where(s < -a, s + a, jnp.zeros_like(s)))


def _dot_t(vt, m, acc=None):
    r = jax.lax.dot_general(
        vt.astype(jnp.bfloat16), m[...],
        dimension_numbers=(((1,), (1,)), ((), ())),
        preferred_element_type=jnp.float32)
    return r if acc is None else r + acc


def _body(x_ref, a_hbm, b_hbm, d_hbm, e_hbm, h_hbm, al_ref, o_ref,
          abuf, dbuf, ebuf, bxt, stage, sem):
    b = pl.program_id(0)

    srcs = (b_hbm, d_hbm, a_hbm, e_hbm, h_hbm)
    dsts = (None, dbuf, abuf, ebuf, None)
    NT = 5 * NCH

    def copy(i):
        bb = b + i // NT
        m, k = divmod(i % NT, NCH)
        return pltpu.make_async_copy(
            srcs[m].at[bb, pl.ds(k * CH, CH), :],
            stage.at[i % NSLOT], sem.at[i % NSLOT])

    def kick(i):
        if i < NT:
            copy(i).start()
        else:
            @pl.when(b + 1 < B)
            def _():
                copy(i).start()

    def land(i, xt=None):
        copy(i).wait()
        m, k = divmod(i, NCH)
        if m == 0:
            bxt[:, pl.ds(k * CH, CH)] = jax.lax.dot_general(
                xt, stage[i % NSLOT],
                dimension_numbers=(((1,), (1,)), ((), ())),
                preferred_element_type=jnp.float32)
        else:
            dsts[m][pl.ds(k * CH, CH), :] = stage[i % NSLOT].astype(jnp.bfloat16)
        kick(i + NSLOT)

    @pl.when(b == 0)
    def _():
        for i in range(NSLOT):
            copy(i).start()

    a = al_ref[0]
    xt = x_ref[0].T
    for i in range(NCH):
        land(i, xt)
    for i in range(NCH, 2 * NCH):
        land(i)
    st = _dot_t(bxt[...], dbuf)
    zt = _soft(st, a)
    for i in range(2 * NCH, 3 * NCH):
        land(i)
    yt = _dot_t(st, abuf, bxt[...])
    for i in range(3 * NCH, 4 * NCH):
        land(i)
    for it in range(NUM_HIDDEN - 1):
        if it > 0:
            yt = _dot_t(st, abuf, bxt[...])
        st = _dot_t(yt, dbuf, _dot_t(zt, ebuf))
        zt = _soft(st, a)
    s = st.T
    for i in range(4 * NCH, NT):
        copy(i).wait()
        k = i - 4 * NCH
        o_ref[0, pl.ds(k * CH, CH), :] = jnp.dot(
            stage[i % NSLOT], s, preferred_element_type=jnp.float32)
        kick(i + NSLOT)


@jax.jit
def kernel(x_c, conv_A, conv_B, conv_D, conv_E, conv_H, alpha):
    return pl.pallas_call(
        _body,
        grid=(B,),
        in_specs=[
            pl.BlockSpec((1, N, F), lambda b: (b, 0, 0)),
            pl.BlockSpec(memory_space=pl.ANY),
            pl.BlockSpec(memory_space=pl.ANY),
            pl.BlockSpec(memory_space=pl.ANY),
            pl.BlockSpec(memory_space=pl.ANY),
            pl.BlockSpec(memory_space=pl.ANY),
            pl.BlockSpec(memory_space=pltpu.SMEM),
        ],
        out_specs=pl.BlockSpec((1, N, F), lambda b: (b, 0, 0)),
        out_shape=jax.ShapeDtypeStruct((B, N, F), jnp.float32),
        scratch_shapes=[
            pltpu.VMEM((N, N), jnp.bfloat16),
            pltpu.VMEM((N, N), jnp.bfloat16),
            pltpu.VMEM((N, N), jnp.bfloat16),
            pltpu.VMEM((F, N), jnp.float32),
            pltpu.VMEM((NSLOT, CH, N), jnp.float32),
            pltpu.SemaphoreType.DMA((NSLOT,)),
        ],
    )(x_c, conv_A, conv_B, conv_D, conv_E, conv_H, alpha)
